# Initial kernel scaffold; baseline (speedup 1.0000x reference)
import jax
import jax.numpy as jnp
from jax.experimental import pallas as pl


def kernel(data, params):
    raise NotImplementedError("write your pallas kernel here")


# speedup vs baseline: 2.6535x; 2.6535x over previous
import functools

import jax
import jax.numpy as jnp
from jax.experimental import pallas as pl
from jax.experimental.pallas import tpu as pltpu

N = 2048
B = 2
KNN = 16
M1, M1P = 409, 512
M2, M2P = 102, 128
KSA = 64
NUM_CLASSES = 13


def _mm(a, b):
    return jnp.dot(a, b, preferred_element_type=jnp.float32)


def _bn(x, g, b, eps):
    return g * x / jnp.sqrt(1.0 + eps) + b


def _lrelu(x):
    return jnp.where(x > 0, x, 0.2 * x)


def _pdist2(a, b):
    an = jnp.sum(a * a, axis=-1)[:, None]
    bn_ = jnp.sum(b * b, axis=-1)[None, :]
    return jnp.maximum(an + bn_ - 2.0 * _mm(a, b.T), 0.0)


def _mlp(x, layers):
    for (W, b, g, be) in layers:
        x = jnp.maximum(_mm(x, W) + b, 0.0)
        x = _bn(x, g, be, 1e-5)
    return x


def _topk_smallest(d2, k):
    R, C = d2.shape
    iota_c = jax.lax.broadcasted_iota(jnp.int32, (R, C), 1)
    iota_k = jax.lax.broadcasted_iota(jnp.int32, (R, k), 1)

    def step(kk, carry):
        cur, vals, idxs = carry
        m = jnp.min(cur, axis=1, keepdims=True)
        first = jnp.min(jnp.where(cur == m, iota_c, C), axis=1,
                        keepdims=True)
        cur = jnp.where(iota_c == first, jnp.float32(jnp.inf), cur)
        vals = jnp.where(iota_k == kk, m, vals)
        idxs = jnp.where(iota_k == kk, first, idxs)
        return cur, vals, idxs

    _, vals, idxs = jax.lax.fori_loop(
        0, k, step,
        (d2, jnp.zeros((R, k), jnp.float32), jnp.zeros((R, k), jnp.int32)))
    return vals, idxs


def _gather_slot(table, idx_col):
    C = table.shape[0]
    R = idx_col.shape[0]
    iota_c = jax.lax.broadcasted_iota(jnp.int32, (R, C), 1)
    oh = (idx_col == iota_c).astype(jnp.float32)
    return _mm(oh, table)


def _fps(pos, nvalid, m, mp):
    P = pos.shape[0]
    iota_p = jax.lax.broadcasted_iota(jnp.int32, (P, 1), 0)
    iota_q = jax.lax.broadcasted_iota(jnp.int32, (mp, 1), 0)
    valid = iota_p < nvalid
    p0 = pos[0:1, :]
    dists = jnp.sum((pos - p0) ** 2, axis=1, keepdims=True)
    dists = jnp.where(valid, dists, -1.0)
    qpos = jnp.where(iota_q == 0, p0, jnp.zeros((mp, 3), jnp.float32))

    def step(i, carry):
        dists, qpos = carry
        mx = jnp.max(dists)
        nxt = jnp.min(jnp.where(dists == mx, iota_p, P))
        prow = jnp.sum(jnp.where(iota_p == nxt, pos, 0.0), axis=0,
                       keepdims=True)
        d = jnp.sum((pos - prow) ** 2, axis=1, keepdims=True)
        dists = jnp.where(valid, jnp.minimum(dists, d), -1.0)
        qpos = jnp.where(iota_q == i, prow, qpos)
        return dists, qpos

    _, qpos = jax.lax.fori_loop(1, m, step, (dists, qpos))
    return qpos


def _sa_stage(x, pos, nvalid, m, mp, r, layers):
    P, C = x.shape
    qpos = _fps(pos, nvalid, m, mp)
    d2 = _pdist2(qpos, pos)
    iota_c = jax.lax.broadcasted_iota(jnp.int32, (1, P), 1)
    d2m = jnp.where(iota_c < nvalid, d2, jnp.float32(jnp.inf))
    vals, idx = _topk_smallest(d2m, KSA)
    tbl = jnp.concatenate([x, pos], axis=1)
    shift = jnp.concatenate([jnp.zeros((mp, C), jnp.float32), qpos], axis=1)
    rows = [_gather_slot(tbl, idx[:, k:k + 1]) - shift for k in range(KSA)]
    h = _mlp(jnp.concatenate(rows, axis=0), layers)
    out = jnp.full((mp, h.shape[1]), -jnp.inf, jnp.float32)
    r2 = r * r
    for k in range(KSA):
        hk = h[k * mp:(k + 1) * mp, :]
        out = jnp.maximum(out, jnp.where(vals[:, k:k + 1] <= r2, hk,
                                         -jnp.inf))
    iota_q = jax.lax.broadcasted_iota(jnp.int32, (mp, 1), 0)
    out = jnp.where(iota_q < m, out, 0.0)
    return out, qpos


def _knn_interp(x_src, pos_src, nsrc, pos_dst, k):
    d2 = _pdist2(pos_dst, pos_src)
    S = pos_src.shape[0]
    iota_c = jax.lax.broadcasted_iota(jnp.int32, (1, S), 1)
    d2m = jnp.where(iota_c < nsrc, d2, jnp.float32(jnp.inf))
    dk, idx = _topk_smallest(d2m, k)
    w = 1.0 / jnp.maximum(dk, 1e-16)
    num = None
    for kk in range(k):
        xg = _gather_slot(x_src, idx[:, kk:kk + 1])
        t = w[:, kk:kk + 1] * xg
        num = t if num is None else num + t
    return num / jnp.sum(w, axis=1)[:, None]


def _body(*refs, treedef):
    data_ref = refs[0]
    out_ref = refs[-1]
    pvals = [r[...] for r in refs[1:-1]]
    p = jax.tree_util.tree_unflatten(treedef, pvals)

    data = data_ref[...].reshape(N, 3 + 3)
    coords = data[:, :3]
    feats = data[:, 3:]

    x = _lrelu(_bn(_mm(feats, p['fc_W']) + p['fc_b'],
                   p['bn0_g'], p['bn0_b'], 1e-6))
    d2 = _pdist2(coords, coords)
    kd2, kidx = _topk_smallest(d2, KNN)
    kdist = jnp.sqrt(jnp.maximum(kd2, 1e-12))
    h = _lrelu(_mm(x, p['mlp1_W']) + p['mlp1_b'])

    lse = []
    zs = []
    for k in range(KNN):
        nb = _gather_slot(coords, kidx[:, k:k + 1])
        spatial = jnp.concatenate(
            [coords, nb, coords - nb, kdist[:, k:k + 1]], axis=1)
        se = jnp.maximum(
            _bn(_mm(spatial, p['lse_W']) + p['lse_b'],
                p['lse_g'], p['lse_be'], 1e-6), 0.0)
        lo = jnp.concatenate([se, h], axis=1)
        lse.append(lo)
        zs.append(_mm(lo, p['pool_score_W']))
    zm = zs[0]
    for k in range(1, KNN):
        zm = jnp.maximum(zm, zs[k])
    ez = [jnp.exp(z - zm) for z in zs]
    es = ez[0]
    for k in range(1, KNN):
        es = es + ez[k]
    pooled = ez[0] / es * lse[0]
    for k in range(1, KNN):
        pooled = pooled + ez[k] / es * lse[k]
    x0 = jnp.maximum(_bn(_mm(pooled, p['pool_W']) + p['pool_b'],
                         p['pool_g'], p['pool_be'], 1e-6), 0.0)

    x1, pos1 = _sa_stage(x0, coords, N, M1, M1P, 0.2, p['sa1'])
    x2, pos2 = _sa_stage(x1, pos1, M1, M2, M2P, 0.4, p['sa2'])

    h3 = _mlp(jnp.concatenate([x2, pos2], axis=1), p['sa3'])
    iota_q2 = jax.lax.broadcasted_iota(jnp.int32, (M2P, 1), 0)
    x3 = jnp.max(jnp.where(iota_q2 < M2, h3, -jnp.inf), axis=0,
                 keepdims=True)

    xi3 = jnp.broadcast_to(x3, (M2P, 1024))
    f3 = _mlp(jnp.concatenate([xi3, x2], axis=1), p['fp3'])
    xi2 = _knn_interp(f3, pos2, M2, pos1, 3)
    f2 = _mlp(jnp.concatenate([xi2, x1], axis=1), p['fp2'])
    xi1 = _knn_interp(f2, pos1, M1, coords, 3)
    f1 = _mlp(jnp.concatenate([xi1, x0], axis=1), p['fp1'])

    y = jnp.maximum(_mm(f1, p['lin1_W']) + p['lin1_b'], 0.0)
    y = _mm(y, p['lin2_W']) + p['lin2_b']
    y = _mm(y, p['lin3_W']) + p['lin3_b']
    ym = jnp.max(y, axis=1, keepdims=True)
    ey = jnp.exp(y - ym)
    y = (y - ym) - jnp.log(jnp.sum(ey, axis=1, keepdims=True))
    out_ref[...] = y.reshape(1, N, NUM_CLASSES)


def kernel(data, params):
    leaves, treedef = jax.tree_util.tree_flatten(params)
    leaves = [l.reshape(1, -1) if l.ndim == 1 else l for l in leaves]
    in_specs = [pl.BlockSpec((1, N, 6), lambda b: (b, 0, 0))]
    for l in leaves:
        in_specs.append(
            pl.BlockSpec(l.shape, functools.partial(lambda nd, b: (0,) * nd,
                                                    l.ndim)))
    out = pl.pallas_call(
        functools.partial(_body, treedef=treedef),
        grid=(B,),
        in_specs=in_specs,
        out_specs=pl.BlockSpec((1, N, NUM_CLASSES), lambda b: (b, 0, 0)),
        out_shape=jax.ShapeDtypeStruct((B, N, NUM_CLASSES), jnp.float32),
        compiler_params=pltpu.CompilerParams(
            dimension_semantics=("arbitrary",),
            vmem_limit_bytes=128 * 1024 * 1024),
    )(data, *leaves)
    return out.reshape(B * N, NUM_CLASSES)


# speedup vs baseline: 4.1808x; 1.5756x over previous
import functools

import jax
import jax.numpy as jnp
from jax.experimental import pallas as pl
from jax.experimental.pallas import tpu as pltpu

N = 2048
B = 2
KNN = 16
M1, M1P = 409, 512
M2, M2P = 102, 128
KSA = 64
NUM_CLASSES = 13


def _mm(a, b):
    return jnp.dot(a, b, preferred_element_type=jnp.float32)


def _bn(x, g, b, eps):
    return g * x / jnp.sqrt(1.0 + eps) + b


def _lrelu(x):
    return jnp.where(x > 0, x, 0.2 * x)


def _pdist2(a, b):
    an = jnp.sum(a * a, axis=-1)[:, None]
    bn_ = jnp.sum(b * b, axis=-1)[None, :]
    return jnp.maximum(an + bn_ - 2.0 * _mm(a, b.T), 0.0)


def _mlp(x, layers):
    for (W, b, g, be) in layers:
        x = jnp.maximum(_mm(x, W) + b, 0.0)
        x = _bn(x, g, be, 1e-5)
    return x


def _topk_smallest(d2, k):
    R, C = d2.shape
    iota_c = jax.lax.broadcasted_iota(jnp.int32, (R, C), 1)
    iota_k = jax.lax.broadcasted_iota(jnp.int32, (R, k), 1)

    def step(kk, carry):
        cur, vals, idxs = carry
        m = jnp.min(cur, axis=1, keepdims=True)
        first = jnp.min(jnp.where(cur == m, iota_c, C), axis=1,
                        keepdims=True)
        cur = jnp.where(iota_c == first, jnp.float32(jnp.inf), cur)
        vals = jnp.where(iota_k == kk, m, vals)
        idxs = jnp.where(iota_k == kk, first, idxs)
        return cur, vals, idxs

    _, vals, idxs = jax.lax.fori_loop(
        0, k, step,
        (d2, jnp.zeros((R, k), jnp.float32), jnp.zeros((R, k), jnp.int32)))
    return vals, idxs


def _gather_slot(table, idx_col):
    C = table.shape[0]
    R = idx_col.shape[0]
    iota_c = jax.lax.broadcasted_iota(jnp.int32, (R, C), 1)
    oh = (idx_col == iota_c).astype(jnp.float32)
    return _mm(oh, table)


def _fps(pos, nvalid, m, mp):
    P = pos.shape[0]
    posT = pos.T
    iota_l = jax.lax.broadcasted_iota(jnp.int32, (1, P), 1)
    iota_q = jax.lax.broadcasted_iota(jnp.int32, (1, mp), 1)
    valid = iota_l < nvalid
    p0 = posT[:, 0:1]
    dists = jnp.sum((posT - p0) ** 2, axis=0, keepdims=True)
    dists = jnp.where(valid, dists, -1.0)
    idxs = jnp.zeros((1, mp), jnp.int32)

    def step(i, carry):
        dists, idxs = carry
        mx = jnp.max(dists)
        nxt = jnp.min(jnp.where(dists == mx, iota_l, P))
        prow = jnp.sum(jnp.where(iota_l == nxt, posT, 0.0), axis=1,
                       keepdims=True)
        d = jnp.sum((posT - prow) ** 2, axis=0, keepdims=True)
        dists = jnp.where(valid, jnp.minimum(dists, d), -1.0)
        idxs = jnp.where(iota_q == i, nxt, idxs)
        return dists, idxs

    _, idxs = jax.lax.fori_loop(1, m, step, (dists, idxs))
    return _gather_slot(pos, idxs.T)


def _sa_stage(x, pos, nvalid, m, mp, r, layers):
    P, C = x.shape
    qpos = _fps(pos, nvalid, m, mp)
    d2 = _pdist2(qpos, pos)
    iota_c = jax.lax.broadcasted_iota(jnp.int32, (1, P), 1)
    d2m = jnp.where(iota_c < nvalid, d2, jnp.float32(jnp.inf))
    vals, idx = _topk_smallest(d2m, KSA)
    tbl = jnp.concatenate([x, pos], axis=1)
    shift = jnp.concatenate([jnp.zeros((mp, C), jnp.float32), qpos], axis=1)
    rows = [_gather_slot(tbl, idx[:, k:k + 1]) - shift for k in range(KSA)]
    h = _mlp(jnp.concatenate(rows, axis=0), layers)
    out = jnp.full((mp, h.shape[1]), -jnp.inf, jnp.float32)
    r2 = r * r
    for k in range(KSA):
        hk = h[k * mp:(k + 1) * mp, :]
        out = jnp.maximum(out, jnp.where(vals[:, k:k + 1] <= r2, hk,
                                         -jnp.inf))
    iota_q = jax.lax.broadcasted_iota(jnp.int32, (mp, 1), 0)
    out = jnp.where(iota_q < m, out, 0.0)
    return out, qpos


def _knn_interp(x_src, pos_src, nsrc, pos_dst, k):
    d2 = _pdist2(pos_dst, pos_src)
    S = pos_src.shape[0]
    iota_c = jax.lax.broadcasted_iota(jnp.int32, (1, S), 1)
    d2m = jnp.where(iota_c < nsrc, d2, jnp.float32(jnp.inf))
    dk, idx = _topk_smallest(d2m, k)
    w = 1.0 / jnp.maximum(dk, 1e-16)
    num = None
    for kk in range(k):
        xg = _gather_slot(x_src, idx[:, kk:kk + 1])
        t = w[:, kk:kk + 1] * xg
        num = t if num is None else num + t
    return num / jnp.sum(w, axis=1)[:, None]


def _body(*refs, treedef):
    data_ref = refs[0]
    out_ref = refs[-1]
    pvals = [r[...] for r in refs[1:-1]]
    p = jax.tree_util.tree_unflatten(treedef, pvals)

    data = data_ref[...].reshape(N, 3 + 3)
    coords = data[:, :3]
    feats = data[:, 3:]

    x = _lrelu(_bn(_mm(feats, p['fc_W']) + p['fc_b'],
                   p['bn0_g'], p['bn0_b'], 1e-6))
    d2 = _pdist2(coords, coords)
    kd2, kidx = _topk_smallest(d2, KNN)
    kdist = jnp.sqrt(jnp.maximum(kd2, 1e-12))
    h = _lrelu(_mm(x, p['mlp1_W']) + p['mlp1_b'])

    lse = []
    zs = []
    for k in range(KNN):
        nb = _gather_slot(coords, kidx[:, k:k + 1])
        spatial = jnp.concatenate(
            [coords, nb, coords - nb, kdist[:, k:k + 1]], axis=1)
        se = jnp.maximum(
            _bn(_mm(spatial, p['lse_W']) + p['lse_b'],
                p['lse_g'], p['lse_be'], 1e-6), 0.0)
        lo = jnp.concatenate([se, h], axis=1)
        lse.append(lo)
        zs.append(_mm(lo, p['pool_score_W']))
    zm = zs[0]
    for k in range(1, KNN):
        zm = jnp.maximum(zm, zs[k])
    ez = [jnp.exp(z - zm) for z in zs]
    es = ez[0]
    for k in range(1, KNN):
        es = es + ez[k]
    pooled = ez[0] / es * lse[0]
    for k in range(1, KNN):
        pooled = pooled + ez[k] / es * lse[k]
    x0 = jnp.maximum(_bn(_mm(pooled, p['pool_W']) + p['pool_b'],
                         p['pool_g'], p['pool_be'], 1e-6), 0.0)

    x1, pos1 = _sa_stage(x0, coords, N, M1, M1P, 0.2, p['sa1'])
    x2, pos2 = _sa_stage(x1, pos1, M1, M2, M2P, 0.4, p['sa2'])

    h3 = _mlp(jnp.concatenate([x2, pos2], axis=1), p['sa3'])
    iota_q2 = jax.lax.broadcasted_iota(jnp.int32, (M2P, 1), 0)
    x3 = jnp.max(jnp.where(iota_q2 < M2, h3, -jnp.inf), axis=0,
                 keepdims=True)

    xi3 = jnp.broadcast_to(x3, (M2P, 1024))
    f3 = _mlp(jnp.concatenate([xi3, x2], axis=1), p['fp3'])
    xi2 = _knn_interp(f3, pos2, M2, pos1, 3)
    f2 = _mlp(jnp.concatenate([xi2, x1], axis=1), p['fp2'])
    xi1 = _knn_interp(f2, pos1, M1, coords, 3)
    f1 = _mlp(jnp.concatenate([xi1, x0], axis=1), p['fp1'])

    y = jnp.maximum(_mm(f1, p['lin1_W']) + p['lin1_b'], 0.0)
    y = _mm(y, p['lin2_W']) + p['lin2_b']
    y = _mm(y, p['lin3_W']) + p['lin3_b']
    ym = jnp.max(y, axis=1, keepdims=True)
    ey = jnp.exp(y - ym)
    y = (y - ym) - jnp.log(jnp.sum(ey, axis=1, keepdims=True))
    out_ref[...] = y.reshape(1, N, NUM_CLASSES)


def kernel(data, params):
    leaves, treedef = jax.tree_util.tree_flatten(params)
    leaves = [l.reshape(1, -1) if l.ndim == 1 else l for l in leaves]
    in_specs = [pl.BlockSpec((1, N, 6), lambda b: (b, 0, 0))]
    for l in leaves:
        in_specs.append(
            pl.BlockSpec(l.shape, functools.partial(lambda nd, b: (0,) * nd,
                                                    l.ndim)))
    out = pl.pallas_call(
        functools.partial(_body, treedef=treedef),
        grid=(B,),
        in_specs=in_specs,
        out_specs=pl.BlockSpec((1, N, NUM_CLASSES), lambda b: (b, 0, 0)),
        out_shape=jax.ShapeDtypeStruct((B, N, NUM_CLASSES), jnp.float32),
        compiler_params=pltpu.CompilerParams(
            dimension_semantics=("arbitrary",),
            vmem_limit_bytes=128 * 1024 * 1024),
    )(data, *leaves)
    return out.reshape(B * N, NUM_CLASSES)


# speedup vs baseline: 5.1597x; 1.2341x over previous
import functools

import jax
import jax.numpy as jnp
from jax.experimental import pallas as pl
from jax.experimental.pallas import tpu as pltpu

N = 2048
B = 2
KNN = 16
M1, M1P = 409, 512
M2, M2P = 102, 128
KSA = 64
NUM_CLASSES = 13


def _mm(a, b):
    return jnp.dot(a, b, preferred_element_type=jnp.float32)


def _bn(x, g, b, eps):
    return g * x / jnp.sqrt(1.0 + eps) + b


def _lrelu(x):
    return jnp.where(x > 0, x, 0.2 * x)


def _pdist2(a, b):
    an = jnp.sum(a * a, axis=-1)[:, None]
    bn_ = jnp.sum(b * b, axis=-1)[None, :]
    return jnp.maximum(an + bn_ - 2.0 * _mm(a, b.T), 0.0)


def _mlp(x, layers):
    for (W, b, g, be) in layers:
        x = jnp.maximum(_mm(x, W) + b, 0.0)
        x = _bn(x, g, be, 1e-5)
    return x


def _topk_smallest(d2, k):
    R, C = d2.shape
    iota_c = jax.lax.broadcasted_iota(jnp.int32, (R, C), 1)
    iota_k = jax.lax.broadcasted_iota(jnp.int32, (R, k), 1)

    def step(kk, carry):
        cur, vals, idxs = carry
        m = jnp.min(cur, axis=1, keepdims=True)
        first = jnp.min(jnp.where(cur == m, iota_c, C), axis=1,
                        keepdims=True)
        cur = jnp.where(iota_c == first, jnp.float32(jnp.inf), cur)
        vals = jnp.where(iota_k == kk, m, vals)
        idxs = jnp.where(iota_k == kk, first, idxs)
        return cur, vals, idxs

    _, vals, idxs = jax.lax.fori_loop(
        0, k, step,
        (d2, jnp.zeros((R, k), jnp.float32), jnp.zeros((R, k), jnp.int32)))
    return vals, idxs


def _gather_slot(table, idx_col):
    C = table.shape[0]
    R = idx_col.shape[0]
    iota_c = jax.lax.broadcasted_iota(jnp.int32, (R, C), 1)
    oh = (idx_col == iota_c).astype(jnp.float32)
    return _mm(oh, table)


def _fps_pair(pos_a, pos_b, nvalid, m, mp):
    P = pos_a.shape[0]
    posT6 = jnp.concatenate([pos_a.T, pos_b.T], axis=0)
    iota_l = jax.lax.broadcasted_iota(jnp.int32, (2, P), 1)
    iota_q = jax.lax.broadcasted_iota(jnp.int32, (2, mp), 1)
    valid = iota_l < nvalid

    def pairdist(sq):
        return jnp.concatenate([sq[0:1] + sq[1:2] + sq[2:3],
                                sq[3:4] + sq[4:5] + sq[5:6]], axis=0)

    dists = pairdist((posT6 - posT6[:, 0:1]) ** 2)
    dists = jnp.where(valid, dists, -1.0)
    idxs = jnp.zeros((2, mp), jnp.int32)

    def step(i, carry):
        dists, idxs = carry
        mx = jnp.max(dists, axis=1, keepdims=True)
        nxt = jnp.min(jnp.where(dists == mx, iota_l, P), axis=1,
                      keepdims=True)
        mask2 = (iota_l == nxt).astype(jnp.float32)
        mask6 = jnp.concatenate([mask2[0:1]] * 3 + [mask2[1:2]] * 3,
                                axis=0)
        prow6 = jnp.sum(posT6 * mask6, axis=1, keepdims=True)
        d = pairdist((posT6 - prow6) ** 2)
        dists = jnp.where(valid, jnp.minimum(dists, d), -1.0)
        idxs = jnp.where(iota_q == i, nxt, idxs)
        return dists, idxs

    _, idxs = jax.lax.fori_loop(1, m, step, (dists, idxs))
    return idxs


def _sa_stage(x, pos, qpos, nvalid, m, mp, r, layers):
    P, C = x.shape
    d2 = _pdist2(qpos, pos)
    iota_c = jax.lax.broadcasted_iota(jnp.int32, (1, P), 1)
    d2m = jnp.where(iota_c < nvalid, d2, jnp.float32(jnp.inf))
    vals, idx = _topk_smallest(d2m, KSA)
    tbl = jnp.concatenate([x, pos], axis=1)
    shift = jnp.concatenate([jnp.zeros((mp, C), jnp.float32), qpos], axis=1)
    rows = [_gather_slot(tbl, idx[:, k:k + 1]) - shift for k in range(KSA)]
    h = _mlp(jnp.concatenate(rows, axis=0), layers)
    out = jnp.full((mp, h.shape[1]), -jnp.inf, jnp.float32)
    r2 = r * r
    for k in range(KSA):
        hk = h[k * mp:(k + 1) * mp, :]
        out = jnp.maximum(out, jnp.where(vals[:, k:k + 1] <= r2, hk,
                                         -jnp.inf))
    iota_q = jax.lax.broadcasted_iota(jnp.int32, (mp, 1), 0)
    out = jnp.where(iota_q < m, out, 0.0)
    return out


def _knn_interp(x_src, pos_src, nsrc, pos_dst, k):
    d2 = _pdist2(pos_dst, pos_src)
    S = pos_src.shape[0]
    iota_c = jax.lax.broadcasted_iota(jnp.int32, (1, S), 1)
    d2m = jnp.where(iota_c < nsrc, d2, jnp.float32(jnp.inf))
    dk, idx = _topk_smallest(d2m, k)
    w = 1.0 / jnp.maximum(dk, 1e-16)
    num = None
    for kk in range(k):
        xg = _gather_slot(x_src, idx[:, kk:kk + 1])
        t = w[:, kk:kk + 1] * xg
        num = t if num is None else num + t
    return num / jnp.sum(w, axis=1)[:, None]


def _lse_stage(coords, feats, p):
    x = _lrelu(_bn(_mm(feats, p['fc_W']) + p['fc_b'],
                   p['bn0_g'], p['bn0_b'], 1e-6))
    d2 = _pdist2(coords, coords)
    kd2, kidx = _topk_smallest(d2, KNN)
    kdist = jnp.sqrt(jnp.maximum(kd2, 1e-12))
    h = _lrelu(_mm(x, p['mlp1_W']) + p['mlp1_b'])

    lse = []
    zs = []
    for k in range(KNN):
        nb = _gather_slot(coords, kidx[:, k:k + 1])
        spatial = jnp.concatenate(
            [coords, nb, coords - nb, kdist[:, k:k + 1]], axis=1)
        se = jnp.maximum(
            _bn(_mm(spatial, p['lse_W']) + p['lse_b'],
                p['lse_g'], p['lse_be'], 1e-6), 0.0)
        lo = jnp.concatenate([se, h], axis=1)
        lse.append(lo)
        zs.append(_mm(lo, p['pool_score_W']))
    zm = zs[0]
    for k in range(1, KNN):
        zm = jnp.maximum(zm, zs[k])
    ez = [jnp.exp(z - zm) for z in zs]
    es = ez[0]
    for k in range(1, KNN):
        es = es + ez[k]
    pooled = ez[0] / es * lse[0]
    for k in range(1, KNN):
        pooled = pooled + ez[k] / es * lse[k]
    return jnp.maximum(_bn(_mm(pooled, p['pool_W']) + p['pool_b'],
                           p['pool_g'], p['pool_be'], 1e-6), 0.0)


def _tail_stages(x0, x1, x2, coords, pos1, pos2, p):
    h3 = _mlp(jnp.concatenate([x2, pos2], axis=1), p['sa3'])
    iota_q2 = jax.lax.broadcasted_iota(jnp.int32, (M2P, 1), 0)
    x3 = jnp.max(jnp.where(iota_q2 < M2, h3, -jnp.inf), axis=0,
                 keepdims=True)
    xi3 = jnp.broadcast_to(x3, (M2P, 1024))
    f3 = _mlp(jnp.concatenate([xi3, x2], axis=1), p['fp3'])
    xi2 = _knn_interp(f3, pos2, M2, pos1, 3)
    f2 = _mlp(jnp.concatenate([xi2, x1], axis=1), p['fp2'])
    xi1 = _knn_interp(f2, pos1, M1, coords, 3)
    f1 = _mlp(jnp.concatenate([xi1, x0], axis=1), p['fp1'])

    y = jnp.maximum(_mm(f1, p['lin1_W']) + p['lin1_b'], 0.0)
    y = _mm(y, p['lin2_W']) + p['lin2_b']
    y = _mm(y, p['lin3_W']) + p['lin3_b']
    ym = jnp.max(y, axis=1, keepdims=True)
    ey = jnp.exp(y - ym)
    return (y - ym) - jnp.log(jnp.sum(ey, axis=1, keepdims=True))


def _body(*refs, treedef):
    data_ref = refs[0]
    out_ref = refs[-1]
    pvals = [r[...] for r in refs[1:-1]]
    p = jax.tree_util.tree_unflatten(treedef, pvals)

    data = data_ref[...]
    coords = []
    x0 = []
    for b in range(B):
        cb = data[b, :, :3]
        coords.append(cb)
        x0.append(_lse_stage(cb, data[b, :, 3:], p))

    idxs1 = _fps_pair(coords[0], coords[1], N, M1, M1P)
    pos1 = []
    x1 = []
    for b in range(B):
        qp = _gather_slot(coords[b], idxs1[b:b + 1, :].T)
        pos1.append(qp)
        x1.append(_sa_stage(x0[b], coords[b], qp, N, M1, M1P, 0.2,
                            p['sa1']))

    idxs2 = _fps_pair(pos1[0], pos1[1], M1, M2, M2P)
    for b in range(B):
        qp2 = _gather_slot(pos1[b], idxs2[b:b + 1, :].T)
        x2 = _sa_stage(x1[b], pos1[b], qp2, M1, M2, M2P, 0.4, p['sa2'])
        y = _tail_stages(x0[b], x1[b], x2, coords[b], pos1[b], qp2, p)
        out_ref[b:b + 1, :, :] = y.reshape(1, N, NUM_CLASSES)


def kernel(data, params):
    leaves, treedef = jax.tree_util.tree_flatten(params)
    leaves = [l.reshape(1, -1) if l.ndim == 1 else l for l in leaves]
    out = pl.pallas_call(
        functools.partial(_body, treedef=treedef),
        out_shape=jax.ShapeDtypeStruct((B, N, NUM_CLASSES), jnp.float32),
        compiler_params=pltpu.CompilerParams(
            vmem_limit_bytes=100 * 1024 * 1024),
    )(data, *leaves)
    return out.reshape(B * N, NUM_CLASSES)
